# baseline (speedup 1.0000x reference)
import jax
import jax.numpy as jnp
from jax.experimental import pallas as pl

_ROW_BLOCK = 256


def _gat_body(x_ref, w_ref, att_s_ref, att_d_ref, bias_ref, out_ref):
    p = x_ref.shape[0]
    dout = w_ref.shape[1]
    att = jnp.stack([att_s_ref[...], att_d_ref[...]], axis=1)
    wa = jnp.dot(w_ref[...], att, preferred_element_type=jnp.float32)
    w_ext = jnp.concatenate([w_ref[...], wa], axis=1)
    hx = jnp.dot(x_ref[...], w_ext, preferred_element_type=jnp.float32)
    s = hx[:, dout]
    d = hx[:, dout + 1]
    h1 = jnp.concatenate(
        [hx[:, :dout], jnp.ones((p, 1), jnp.float32)],
        axis=1).astype(jnp.bfloat16)
    bias = bias_ref[...].reshape(1, dout)
    b = min(_ROW_BLOCK, p)
    row = jax.lax.broadcasted_iota(jnp.int32, (b, b), 0)
    col = jax.lax.broadcasted_iota(jnp.int32, (b, b), 1)
    tri = col <= row

    def leaky(v):
        return jnp.maximum(v, 0.2 * v)

    for r0 in range(0, p, _ROW_BLOCK):
        rn = min(_ROW_BLOCK, p - r0)
        db = d[r0:r0 + rn][:, None]
        exd = jnp.exp(leaky(db + s[None, r0:r0 + rn])).astype(jnp.bfloat16)
        exd = jnp.where(tri[:rn, :rn], exd, jnp.bfloat16(0.0))
        acc = jnp.dot(exd, h1[r0:r0 + rn],
                      preferred_element_type=jnp.float32)
        if r0 > 0:
            exl = jnp.exp(leaky(db + s[None, :r0])).astype(jnp.bfloat16)
            acc = acc + jnp.dot(exl, h1[:r0],
                                preferred_element_type=jnp.float32)
        out = acc[:, :dout] / acc[:, dout:dout + 1] + bias
        out_ref[r0:r0 + rn, :] = jnp.maximum(out, 0.0)


def kernel(x, W, att_src, att_dst, bias):
    n, _ = x.shape
    dout = W.shape[1]
    return pl.pallas_call(
        _gat_body,
        out_shape=jax.ShapeDtypeStruct((n, dout), jnp.float32),
    )(x, W, att_src, att_dst, bias)


# speedup vs baseline: 1.0724x; 1.0724x over previous
import jax
import jax.numpy as jnp
from jax.experimental import pallas as pl

_ROW_BLOCK = 256


def _gat_body(x_ref, w_ref, att_s_ref, att_d_ref, bias_ref, out_ref):
    p = x_ref.shape[0]
    dout = w_ref.shape[1]
    att2 = jnp.concatenate(
        [att_s_ref[...].reshape(1, dout), att_d_ref[...].reshape(1, dout)],
        axis=0) * jnp.float32(1.4426950408889634)
    wa = jax.lax.dot_general(
        w_ref[...], att2, (((1,), (1,)), ((), ())),
        preferred_element_type=jnp.float32)
    w_ext = jnp.concatenate([w_ref[...], wa], axis=1)
    hx = jnp.dot(x_ref[...], w_ext, preferred_element_type=jnp.float32)
    s = hx[:, dout]
    d = hx[:, dout + 1]
    h1 = jnp.concatenate(
        [hx[:, :dout], jnp.ones((p, 1), jnp.float32)],
        axis=1)
    bias = bias_ref[...].reshape(1, dout)
    b = min(_ROW_BLOCK, p)
    row = jax.lax.broadcasted_iota(jnp.int32, (b, b), 0)
    col = jax.lax.broadcasted_iota(jnp.int32, (b, b), 1)
    tri = col <= row

    def leaky(v):
        return jnp.maximum(v, 0.2 * v)

    for r0 in range(0, p, _ROW_BLOCK):
        rn = min(_ROW_BLOCK, p - r0)
        db = d[r0:r0 + rn][:, None]
        exd = jnp.exp2(leaky(db + s[None, r0:r0 + rn]))
        exd = jnp.where(tri[:rn, :rn], exd, 0.0)
        acc = jnp.dot(exd, h1[r0:r0 + rn],
                      preferred_element_type=jnp.float32)
        if r0 > 0:
            exl = jnp.exp2(leaky(db + s[None, :r0]))
            acc = acc + jnp.dot(exl, h1[:r0],
                                preferred_element_type=jnp.float32)
        out = acc[:, :dout] / acc[:, dout:dout + 1] + bias
        out_ref[r0:r0 + rn, :] = jnp.maximum(out, 0.0)


def kernel(x, W, att_src, att_dst, bias):
    n, _ = x.shape
    dout = W.shape[1]
    return pl.pallas_call(
        _gat_body,
        out_shape=jax.ShapeDtypeStruct((n, dout), jnp.float32),
    )(x, W, att_src, att_dst, bias)


# speedup vs baseline: 1.0844x; 1.0112x over previous
import jax
import jax.numpy as jnp
from jax.experimental import pallas as pl

_ROW_BLOCK = 256


def _gat_body(x_ref, w_ref, att_s_ref, att_d_ref, bias_ref, out_ref):
    p = x_ref.shape[0]
    dout = w_ref.shape[1]
    log2e = jnp.float32(1.4426950408889634)
    atts = att_s_ref[...].reshape(1, dout) * log2e
    attd = att_d_ref[...].reshape(1, dout) * log2e
    w = w_ref[...]
    wa_s = jnp.sum(w * atts, axis=1, keepdims=True)
    wa_d = jnp.sum(w * attd, axis=1, keepdims=True)
    w_ext = jnp.concatenate([w, wa_s, wa_d], axis=1)
    hx = jnp.dot(x_ref[...], w_ext, preferred_element_type=jnp.float32)
    s = hx[:, dout]
    d = hx[:, dout + 1]
    h1 = jnp.concatenate(
        [hx[:, :dout], jnp.ones((p, 1), jnp.float32)],
        axis=1)
    bias = bias_ref[...].reshape(1, dout)
    b = min(_ROW_BLOCK, p)
    row = jax.lax.broadcasted_iota(jnp.int32, (b, b), 0)
    col = jax.lax.broadcasted_iota(jnp.int32, (b, b), 1)
    tri = col <= row

    def leaky(v):
        return jnp.maximum(v, 0.2 * v)

    for r0 in range(0, p, _ROW_BLOCK):
        rn = min(_ROW_BLOCK, p - r0)
        db = d[r0:r0 + rn][:, None]
        exd = jnp.exp2(leaky(db + s[None, r0:r0 + rn]))
        exd = jnp.where(tri[:rn, :rn], exd, 0.0)
        acc = jnp.dot(exd, h1[r0:r0 + rn],
                      preferred_element_type=jnp.float32)
        if r0 > 0:
            exl = jnp.exp2(leaky(db + s[None, :r0]))
            acc = acc + jnp.dot(exl, h1[:r0],
                                preferred_element_type=jnp.float32)
        inv = 1.0 / acc[:, dout:dout + 1]
        out = acc[:, :dout] * inv + bias
        out_ref[r0:r0 + rn, :] = jnp.maximum(out, 0.0)


def kernel(x, W, att_src, att_dst, bias):
    n, _ = x.shape
    dout = W.shape[1]
    return pl.pallas_call(
        _gat_body,
        out_shape=jax.ShapeDtypeStruct((n, dout), jnp.float32),
    )(x, W, att_src, att_dst, bias)
